# Initial kernel scaffold; baseline (speedup 1.0000x reference)
import jax
import jax.numpy as jnp
from jax.experimental import pallas as pl


def kernel(sample, ent_embeddings, rel_embeddings, ent_transfer, rel_transfer):
    raise NotImplementedError("write your pallas kernel here")


# speedup vs baseline: 1.4564x; 1.4564x over previous
import functools

import jax
import jax.numpy as jnp
from jax import lax
from jax.experimental import pallas as pl
from jax.experimental.pallas import tpu as pltpu
from jax.experimental.pallas import tpu_sc as plsc

_B = 16384
_D = 128
_MARGIN = 1.0
_NC = 2
_NS = 16
_NW = _NC * _NS
_BPW = _B // _NW
_C = 128
_NCH = _BPW // _C


def _gather6(h_ids, r_ids, t_ids, ent_emb, rel_emb, ent_tr, rel_tr):
    mesh = plsc.VectorSubcoreMesh(
        core_axis_name="c", subcore_axis_name="s",
        num_cores=_NC, num_subcores=_NS)
    row = jax.ShapeDtypeStruct((_B, _D), jnp.float32)

    def body(h_ref, r_ref, t_ref, ee_ref, re_ref, et_ref, rt_ref,
             oh, orl, ot, ohtr, ortr, ottr,
             hidx, ridx, tidx, bufs, sem):
        wid = lax.axis_index("s") * _NC + lax.axis_index("c")
        base = wid * _BPW
        pltpu.sync_copy(h_ref.at[pl.ds(base, _BPW)], hidx)
        pltpu.sync_copy(r_ref.at[pl.ds(base, _BPW)], ridx)
        pltpu.sync_copy(t_ref.at[pl.ds(base, _BPW)], tidx)
        jobs = ((ee_ref, hidx, oh), (re_ref, ridx, orl), (ee_ref, tidx, ot),
                (et_ref, hidx, ohtr), (rt_ref, ridx, ortr), (et_ref, tidx, ottr))

        def chunk(c, carry):
            off = c * _C
            copies = []
            for j, (tbl, idx, _) in enumerate(jobs):
                copies.append(
                    pltpu.async_copy(tbl.at[idx.at[pl.ds(off, _C)]],
                                     bufs.at[j], sem))
            for cp in copies:
                cp.wait()
            for j, (_, _, out) in enumerate(jobs):
                pltpu.sync_copy(bufs.at[j], out.at[pl.ds(base + off, _C)])
            return carry

        lax.fori_loop(0, _NCH, chunk, 0)

    fn = pl.kernel(
        body,
        out_type=(row,) * 6,
        mesh=mesh,
        scratch_types=[
            pltpu.VMEM((_BPW,), jnp.int32),
            pltpu.VMEM((_BPW,), jnp.int32),
            pltpu.VMEM((_BPW,), jnp.int32),
            pltpu.VMEM((6, _C, _D), jnp.float32),
            pltpu.SemaphoreType.DMA,
        ],
    )
    return fn(h_ids, r_ids, t_ids, ent_emb, rel_emb, ent_tr, rel_tr)


def _l2n(x):
    n = jnp.sqrt(jnp.sum(x * x, axis=-1, keepdims=True))
    return x / jnp.maximum(n, 1e-12)


def _score_body(h_ref, r_ref, t_ref, htr_ref, rtr_ref, ttr_ref, o_ref):
    head = h_ref[...]
    rel = r_ref[...]
    tail = t_ref[...]
    h_tr = htr_ref[...]
    r_tr = rtr_ref[...]
    t_tr = ttr_ref[...]
    hh = _l2n(head + jnp.sum(head * h_tr, axis=-1, keepdims=True) * r_tr)
    tt = _l2n(tail + jnp.sum(tail * t_tr, axis=-1, keepdims=True) * r_tr)
    hh = _l2n(hh)
    rr = _l2n(rel)
    tt = _l2n(tt)
    o_ref[...] = _MARGIN - jnp.sum(jnp.abs(hh + rr - tt), axis=-1,
                                   keepdims=True)


_ROWS_PER_BLOCK = 1024
_GRID = _B // _ROWS_PER_BLOCK


def _score(gh, gr, gt, ghtr, grtr, gttr):
    in_spec = pl.BlockSpec((_ROWS_PER_BLOCK, _D), lambda i: (i, 0))
    out = pl.pallas_call(
        _score_body,
        grid=(_GRID,),
        in_specs=[in_spec] * 6,
        out_specs=pl.BlockSpec((_ROWS_PER_BLOCK, 1), lambda i: (i, 0)),
        out_shape=jax.ShapeDtypeStruct((_B, 1), jnp.float32),
    )(gh, gr, gt, ghtr, grtr, gttr)
    return out.reshape(-1)


def kernel(sample, ent_embeddings, rel_embeddings, ent_transfer, rel_transfer):
    h_ids = sample[:, 0]
    r_ids = sample[:, 1]
    t_ids = sample[:, 2]
    gh, gr, gt, ghtr, grtr, gttr = _gather6(
        h_ids, r_ids, t_ids,
        ent_embeddings, rel_embeddings, ent_transfer, rel_transfer)
    return _score(gh, gr, gt, ghtr, grtr, gttr)


# speedup vs baseline: 1.5531x; 1.0664x over previous
import functools

import jax
import jax.numpy as jnp
from jax import lax
from jax.experimental import pallas as pl
from jax.experimental.pallas import tpu as pltpu
from jax.experimental.pallas import tpu_sc as plsc

_B = 16384
_D = 128
_MARGIN = 1.0
_NC = 2
_NS = 16
_NW = _NC * _NS
_BPW = _B // _NW
_C = 64
_NCH = _BPW // _C
_NPAIR = _NCH // 2


def _gather6(h_ids, r_ids, t_ids, ent_emb, rel_emb, ent_tr, rel_tr):
    mesh = plsc.VectorSubcoreMesh(
        core_axis_name="c", subcore_axis_name="s",
        num_cores=_NC, num_subcores=_NS)
    row = jax.ShapeDtypeStruct((_B, _D), jnp.float32)

    def body(h_ref, r_ref, t_ref, ee_ref, re_ref, et_ref, rt_ref,
             oh, orl, ot, ohtr, ortr, ottr,
             hidx, ridx, tidx, bufs, gs0, gs1, ws0, ws1):
        wid = lax.axis_index("s") * _NC + lax.axis_index("c")
        base = wid * _BPW
        pltpu.sync_copy(h_ref.at[pl.ds(base, _BPW)], hidx)
        pltpu.sync_copy(r_ref.at[pl.ds(base, _BPW)], ridx)
        pltpu.sync_copy(t_ref.at[pl.ds(base, _BPW)], tidx)
        jobs = ((ee_ref, hidx, oh), (re_ref, ridx, orl), (ee_ref, tidx, ot),
                (et_ref, hidx, ohtr), (rt_ref, ridx, ortr), (et_ref, tidx, ottr))

        def g_desc(c, p, sem):
            off = c * _C
            return [pltpu.make_async_copy(tbl.at[idx.at[pl.ds(off, _C)]],
                                          bufs.at[p, j], sem)
                    for j, (tbl, idx, _) in enumerate(jobs)]

        def wb_desc(c, p, sem):
            off = c * _C
            return [pltpu.make_async_copy(bufs.at[p, j],
                                          out.at[pl.ds(base + off, _C)], sem)
                    for j, (_, _, out) in enumerate(jobs)]

        for d in g_desc(0, 0, gs0):
            d.start()

        def pair(k, carry):
            c0 = 2 * k
            c1 = c0 + 1
            for d in g_desc(c0, 0, gs0):
                d.wait()

            @pl.when(k > 0)
            def _():
                for d in wb_desc(c1 - 2, 1, ws1):
                    d.wait()

            for d in g_desc(c1, 1, gs1):
                d.start()
            for d in wb_desc(c0, 0, ws0):
                d.start()
            for d in g_desc(c1, 1, gs1):
                d.wait()

            @pl.when(k + 1 < _NPAIR)
            def _():
                for d in wb_desc(c0, 0, ws0):
                    d.wait()
                for d in g_desc(c0 + 2, 0, gs0):
                    d.start()

            for d in wb_desc(c1, 1, ws1):
                d.start()
            return carry

        lax.fori_loop(0, _NPAIR, pair, 0)
        for d in wb_desc(_NCH - 2, 0, ws0):
            d.wait()
        for d in wb_desc(_NCH - 1, 1, ws1):
            d.wait()

    fn = pl.kernel(
        body,
        out_type=(row,) * 6,
        mesh=mesh,
        scratch_types=[
            pltpu.VMEM((_BPW,), jnp.int32),
            pltpu.VMEM((_BPW,), jnp.int32),
            pltpu.VMEM((_BPW,), jnp.int32),
            pltpu.VMEM((2, 6, _C, _D), jnp.float32),
            pltpu.SemaphoreType.DMA,
            pltpu.SemaphoreType.DMA,
            pltpu.SemaphoreType.DMA,
            pltpu.SemaphoreType.DMA,
        ],
    )
    return fn(h_ids, r_ids, t_ids, ent_emb, rel_emb, ent_tr, rel_tr)


def _l2n(x):
    n = jnp.sqrt(jnp.sum(x * x, axis=-1, keepdims=True))
    return x / jnp.maximum(n, 1e-12)


def _score_body(h_ref, r_ref, t_ref, htr_ref, rtr_ref, ttr_ref, o_ref):
    head = h_ref[...]
    rel = r_ref[...]
    tail = t_ref[...]
    h_tr = htr_ref[...]
    r_tr = rtr_ref[...]
    t_tr = ttr_ref[...]
    hh = _l2n(head + jnp.sum(head * h_tr, axis=-1, keepdims=True) * r_tr)
    tt = _l2n(tail + jnp.sum(tail * t_tr, axis=-1, keepdims=True) * r_tr)
    rr = _l2n(rel)
    o_ref[...] = _MARGIN - jnp.sum(jnp.abs(hh + rr - tt), axis=-1)


_ROWS_PER_BLOCK = 1024
_GRID = _B // _ROWS_PER_BLOCK


def _score(gh, gr, gt, ghtr, grtr, gttr):
    in_spec = pl.BlockSpec((_ROWS_PER_BLOCK, _D), lambda i: (i, 0))
    return pl.pallas_call(
        _score_body,
        grid=(_GRID,),
        in_specs=[in_spec] * 6,
        out_specs=pl.BlockSpec((_ROWS_PER_BLOCK,), lambda i: (i,)),
        out_shape=jax.ShapeDtypeStruct((_B,), jnp.float32),
    )(gh, gr, gt, ghtr, grtr, gttr)


def kernel(sample, ent_embeddings, rel_embeddings, ent_transfer, rel_transfer):
    h_ids = sample[:, 0]
    r_ids = sample[:, 1]
    t_ids = sample[:, 2]
    gh, gr, gt, ghtr, grtr, gttr = _gather6(
        h_ids, r_ids, t_ids,
        ent_embeddings, rel_embeddings, ent_transfer, rel_transfer)
    return _score(gh, gr, gt, ghtr, grtr, gttr)


# speedup vs baseline: 1.5929x; 1.0256x over previous
import functools

import jax
import jax.numpy as jnp
from jax import lax
from jax.experimental import pallas as pl
from jax.experimental.pallas import tpu as pltpu
from jax.experimental.pallas import tpu_sc as plsc

_B = 16384
_D = 128
_MARGIN = 1.0
_NC = 2
_NS = 16
_NW = _NC * _NS
_NSLICE = 2
_BS = _B // _NSLICE
_BPW = _BS // _NW
_C = 64
_NCH = _BPW // _C
_NPAIR = _NCH // 2


def _gather6(h_ids, r_ids, t_ids, ent_emb, rel_emb, ent_tr, rel_tr):
    mesh = plsc.VectorSubcoreMesh(
        core_axis_name="c", subcore_axis_name="s",
        num_cores=_NC, num_subcores=_NS)
    row = jax.ShapeDtypeStruct((_BS, _D), jnp.float32)

    def body(h_ref, r_ref, t_ref, ee_ref, re_ref, et_ref, rt_ref,
             oh, orl, ot, ohtr, ortr, ottr,
             hidx, ridx, tidx, bufs, gs0, gs1, ws0, ws1):
        wid = lax.axis_index("s") * _NC + lax.axis_index("c")
        base = wid * _BPW
        pltpu.sync_copy(h_ref.at[pl.ds(base, _BPW)], hidx)
        pltpu.sync_copy(r_ref.at[pl.ds(base, _BPW)], ridx)
        pltpu.sync_copy(t_ref.at[pl.ds(base, _BPW)], tidx)
        jobs = ((ee_ref, hidx, oh), (re_ref, ridx, orl), (ee_ref, tidx, ot),
                (et_ref, hidx, ohtr), (rt_ref, ridx, ortr), (et_ref, tidx, ottr))

        def g_desc(c, p, sem):
            off = c * _C
            return [pltpu.make_async_copy(tbl.at[idx.at[pl.ds(off, _C)]],
                                          bufs.at[p, j], sem)
                    for j, (tbl, idx, _) in enumerate(jobs)]

        def wb_desc(c, p, sem):
            off = c * _C
            return [pltpu.make_async_copy(bufs.at[p, j],
                                          out.at[pl.ds(base + off, _C)], sem)
                    for j, (_, _, out) in enumerate(jobs)]

        for d in g_desc(0, 0, gs0):
            d.start()

        def pair(k, carry):
            c0 = 2 * k
            c1 = c0 + 1
            for d in g_desc(c0, 0, gs0):
                d.wait()

            @pl.when(k > 0)
            def _():
                for d in wb_desc(c1 - 2, 1, ws1):
                    d.wait()

            for d in g_desc(c1, 1, gs1):
                d.start()
            for d in wb_desc(c0, 0, ws0):
                d.start()
            for d in g_desc(c1, 1, gs1):
                d.wait()

            @pl.when(k + 1 < _NPAIR)
            def _():
                for d in wb_desc(c0, 0, ws0):
                    d.wait()
                for d in g_desc(c0 + 2, 0, gs0):
                    d.start()

            for d in wb_desc(c1, 1, ws1):
                d.start()
            return carry

        lax.fori_loop(0, _NPAIR, pair, 0)
        for d in wb_desc(_NCH - 2, 0, ws0):
            d.wait()
        for d in wb_desc(_NCH - 1, 1, ws1):
            d.wait()

    fn = pl.kernel(
        body,
        out_type=(row,) * 6,
        mesh=mesh,
        scratch_types=[
            pltpu.VMEM((_BPW,), jnp.int32),
            pltpu.VMEM((_BPW,), jnp.int32),
            pltpu.VMEM((_BPW,), jnp.int32),
            pltpu.VMEM((2, 6, _C, _D), jnp.float32),
            pltpu.SemaphoreType.DMA,
            pltpu.SemaphoreType.DMA,
            pltpu.SemaphoreType.DMA,
            pltpu.SemaphoreType.DMA,
        ],
    )
    return fn(h_ids, r_ids, t_ids, ent_emb, rel_emb, ent_tr, rel_tr)


def _l2n(x):
    n = jnp.sqrt(jnp.sum(x * x, axis=-1, keepdims=True))
    return x / jnp.maximum(n, 1e-12)


def _score_body(h_ref, r_ref, t_ref, htr_ref, rtr_ref, ttr_ref, o_ref):
    head = h_ref[...]
    rel = r_ref[...]
    tail = t_ref[...]
    h_tr = htr_ref[...]
    r_tr = rtr_ref[...]
    t_tr = ttr_ref[...]
    hh = _l2n(head + jnp.sum(head * h_tr, axis=-1, keepdims=True) * r_tr)
    tt = _l2n(tail + jnp.sum(tail * t_tr, axis=-1, keepdims=True) * r_tr)
    rr = _l2n(rel)
    o_ref[...] = _MARGIN - jnp.sum(jnp.abs(hh + rr - tt), axis=-1)


_ROWS_PER_BLOCK = 1024
_GRID = _BS // _ROWS_PER_BLOCK


def _score(gh, gr, gt, ghtr, grtr, gttr):
    in_spec = pl.BlockSpec((_ROWS_PER_BLOCK, _D), lambda i: (i, 0))
    return pl.pallas_call(
        _score_body,
        grid=(_GRID,),
        in_specs=[in_spec] * 6,
        out_specs=pl.BlockSpec((_ROWS_PER_BLOCK,), lambda i: (i,)),
        out_shape=jax.ShapeDtypeStruct((_BS,), jnp.float32),
    )(gh, gr, gt, ghtr, grtr, gttr)


def kernel(sample, ent_embeddings, rel_embeddings, ent_transfer, rel_transfer):
    h_ids = sample[:, 0]
    r_ids = sample[:, 1]
    t_ids = sample[:, 2]
    scores = []
    for s in range(_NSLICE):
        lo = s * _BS
        g = _gather6(
            jax.lax.dynamic_slice(h_ids, (lo,), (_BS,)),
            jax.lax.dynamic_slice(r_ids, (lo,), (_BS,)),
            jax.lax.dynamic_slice(t_ids, (lo,), (_BS,)),
            ent_embeddings, rel_embeddings, ent_transfer, rel_transfer)
        scores.append(_score(*g))
    return jnp.concatenate(scores)
